# baseline (speedup 1.0000x reference)
import functools

import jax
import jax.numpy as jnp
from jax import lax
from jax.experimental import pallas as pl
from jax.experimental.pallas import tpu as pltpu
from jax.experimental.pallas import tpu_sc as plsc

LAT = 16
HLF = 8
CH = 128


def _tc_detile(*tables):
    V = tables[0].shape[0]
    W = 16384
    grid = (pl.cdiv(V, W),)
    n = len(tables)
    views = [jnp.transpose(t).reshape(LAT, 1, V) for t in tables]

    def body(*refs):
        for t in range(n):
            for f in range(HLF):
                lo = refs[t * LAT + f][0, 0, :].astype(jnp.bfloat16)
                hi = refs[t * LAT + HLF + f][0, 0, :].astype(jnp.bfloat16)
                lo32 = jax.lax.bitcast_convert_type(
                    lo, jnp.uint16).astype(jnp.uint32)
                hi32 = jax.lax.bitcast_convert_type(
                    hi, jnp.uint16).astype(jnp.uint32)
                refs[n * LAT + t * HLF + f][...] = lo32 | (hi32 << 16)

    in_specs = []
    for _ in range(n):
        for f in range(LAT):
            in_specs.append(
                pl.BlockSpec((1, 1, W), lambda i, f=f: (f, 0, i)))
    return pl.pallas_call(
        body,
        grid=grid,
        in_specs=in_specs,
        out_specs=[pl.BlockSpec((W,), lambda i: (i,))] * (n * HLF),
        out_shape=[jax.ShapeDtypeStruct((V,), jnp.uint32)] * (n * HLF),
    )(*[v for v in views for _ in range(LAT)])


def _sc_gather_t(user_idx, item_idx, slabs):
    B = user_idx.shape[0]
    info = plsc.get_sparse_core_info()
    NC, NS = info.num_cores, info.num_subcores
    NW = NC * NS
    bpw = B // NW
    nch = bpw // CH
    mesh = plsc.VectorSubcoreMesh(core_axis_name="c", subcore_axis_name="s")

    @functools.partial(
        pl.kernel,
        mesh=mesh,
        out_type=tuple(
            jax.ShapeDtypeStruct((HLF, B // CH, CH), jnp.uint32)
            for _ in range(4)),
        scratch_types=[
            pltpu.VMEM((nch, CH), jnp.int32),
            pltpu.VMEM((nch, CH), jnp.int32),
            pltpu.VMEM((HLF, nch, CH), jnp.uint32),
            pltpu.VMEM((HLF, nch, CH), jnp.uint32),
            pltpu.VMEM((HLF, nch, CH), jnp.uint32),
            pltpu.VMEM((HLF, nch, CH), jnp.uint32),
            pltpu.SemaphoreType.DMA,
        ],
    )
    def k(uidx_hbm, iidx_hbm, *rest):
        slab_refs = rest[:4 * HLF]
        outs = rest[4 * HLF:4 * HLF + 4]
        uidx_v, iidx_v, ru, ri, rum, rim, sem = rest[4 * HLF + 4:]
        wid = lax.axis_index("s") * NC + lax.axis_index("c")
        base = wid * bpw
        for c in range(nch):
            pltpu.sync_copy(uidx_hbm.at[pl.ds(base + c * CH, CH)],
                            uidx_v.at[c])
            pltpu.sync_copy(iidx_hbm.at[pl.ds(base + c * CH, CH)],
                            iidx_v.at[c])
        pending = []
        for t, (buf, idxref) in enumerate(((ru, uidx_v), (ri, iidx_v),
                                           (rum, uidx_v), (rim, iidx_v))):
            issued = []
            for f in range(HLF):
                slab = slab_refs[t * HLF + f]
                for c in range(nch):
                    issued.append(pltpu.async_copy(
                        slab.at[idxref.at[c]], buf.at[f, c], sem))
            for cp in pending:
                cp.wait()
            pending = issued
        for cp in pending:
            cp.wait()
        cols = pl.ds(wid * nch, nch)
        for buf, out in zip((ru, ri, rum, rim), outs):
            pltpu.sync_copy(buf, out.at[:, cols])

    outs = k(user_idx, item_idx, *slabs)
    return [o.reshape(HLF, B) for o in outs]


def _unpack(x):
    lo = jax.lax.bitcast_convert_type(
        (x & jnp.uint32(0xFFFF)).astype(jnp.uint16), jnp.bfloat16)
    hi = jax.lax.bitcast_convert_type(
        (x >> 16).astype(jnp.uint16), jnp.bfloat16)
    return lo.astype(jnp.float32), hi.astype(jnp.float32)


def _tc_mlp_t(uT, iT, umfT, imfT, W1, b1, W2, b2, W_out, b_out):
    B = uT.shape[1]
    BLK = 2048
    HID = LAT // 2
    W1aT = W1[:LAT].T
    W1bT = W1[LAT:].T
    W2T = W2.T
    b1c = b1.reshape(LAT, 1)
    b2c = b2.reshape(HID, 1)
    wh2 = W_out[:HID, 0].reshape(1, HID)
    wmf = W_out[HID:, 0].reshape(1, LAT)
    bor = b_out.reshape(1, 1)
    W1a_lo, W1a_hi = W1aT[:, :HLF], W1aT[:, HLF:]
    W1b_lo, W1b_hi = W1bT[:, :HLF], W1bT[:, HLF:]
    wmf_lo, wmf_hi = wmf[:, :HLF], wmf[:, HLF:]

    def body(u_ref, i_ref, umf_ref, imf_ref, w1al, w1ah, w1bl, w1bh, b1_,
             w2, b2_, wh2_, wmfl, wmfh, bo, out_ref):
        ulo, uhi = _unpack(u_ref[...])
        ilo, ihi = _unpack(i_ref[...])
        umlo, umhi = _unpack(umf_ref[...])
        imlo, imhi = _unpack(imf_ref[...])
        h1 = jnp.maximum(
            jnp.dot(w1al[...], ulo, preferred_element_type=jnp.float32)
            + jnp.dot(w1ah[...], uhi, preferred_element_type=jnp.float32)
            + jnp.dot(w1bl[...], ilo, preferred_element_type=jnp.float32)
            + jnp.dot(w1bh[...], ihi, preferred_element_type=jnp.float32)
            + b1_[...], 0.0)
        h2 = jnp.maximum(
            jnp.dot(w2[...], h1, preferred_element_type=jnp.float32)
            + b2_[...], 0.0)
        logit = (jnp.dot(wh2_[...], h2, preferred_element_type=jnp.float32)
                 + jnp.dot(wmfl[...], umlo * imlo,
                           preferred_element_type=jnp.float32)
                 + jnp.dot(wmfh[...], umhi * imhi,
                           preferred_element_type=jnp.float32)
                 + bo[...])
        out_ref[...] = jax.nn.sigmoid(logit)

    col = lambda i: (0, i)
    rep = lambda i: (0, 0)
    return pl.pallas_call(
        body,
        grid=(B // BLK,),
        in_specs=[
            pl.BlockSpec((HLF, BLK), col),
            pl.BlockSpec((HLF, BLK), col),
            pl.BlockSpec((HLF, BLK), col),
            pl.BlockSpec((HLF, BLK), col),
            pl.BlockSpec((LAT, HLF), rep),
            pl.BlockSpec((LAT, HLF), rep),
            pl.BlockSpec((LAT, HLF), rep),
            pl.BlockSpec((LAT, HLF), rep),
            pl.BlockSpec((LAT, 1), rep),
            pl.BlockSpec((HID, LAT), rep),
            pl.BlockSpec((HID, 1), rep),
            pl.BlockSpec((1, HID), rep),
            pl.BlockSpec((1, HLF), rep),
            pl.BlockSpec((1, HLF), rep),
            pl.BlockSpec((1, 1), rep),
        ],
        out_specs=pl.BlockSpec((1, BLK), col),
        out_shape=jax.ShapeDtypeStruct((1, B), jnp.float32),
    )(uT, iT, umfT, imfT, W1a_lo, W1a_hi, W1b_lo, W1b_hi, b1c, W2T, b2c,
      wh2, wmf_lo, wmf_hi, bor)


def kernel(user_indices, item_indices, emb_user_mlp, emb_item_mlp,
           emb_user_mf, emb_item_mf, W1, b1, W2, b2, W_out, b_out):
    slabs = _tc_detile(emb_user_mlp, emb_item_mlp, emb_user_mf, emb_item_mf)
    uT, iT, umfT, imfT = _sc_gather_t(user_indices, item_indices, slabs)
    out = _tc_mlp_t(uT, iT, umfT, imfT, W1, b1, W2, b2, W_out, b_out)
    return out.reshape(user_indices.shape[0], 1)


# speedup vs baseline: 1.3832x; 1.3832x over previous
import functools

import jax
import jax.numpy as jnp
from jax import lax
from jax.experimental import pallas as pl
from jax.experimental.pallas import tpu as pltpu
from jax.experimental.pallas import tpu_sc as plsc

LAT = 16
CH = 128


def _tc_detile(*tables):
    V = tables[0].shape[0]
    W = 16384
    grid = (pl.cdiv(V, W),)
    n = len(tables)
    views = [jnp.transpose(t).reshape(LAT, 1, V) for t in tables]

    def body(*refs):
        for s in range(n * LAT):
            refs[n * LAT + s][...] = refs[s][0, 0, :]

    in_specs = []
    for _ in range(n):
        for f in range(LAT):
            in_specs.append(
                pl.BlockSpec((1, 1, W), lambda i, f=f: (f, 0, i)))
    return pl.pallas_call(
        body,
        grid=grid,
        in_specs=in_specs,
        out_specs=[pl.BlockSpec((W,), lambda i: (i,))] * (n * LAT),
        out_shape=[jax.ShapeDtypeStruct((V,), jnp.float32)] * (n * LAT),
    )(*[v for v in views for _ in range(LAT)])


def _sc_gather_t(user_idx, item_idx, slabs):
    B = user_idx.shape[0]
    info = plsc.get_sparse_core_info()
    NC, NS = info.num_cores, info.num_subcores
    NW = NC * NS
    bpw = B // NW
    nch = bpw // CH
    mesh = plsc.VectorSubcoreMesh(core_axis_name="c", subcore_axis_name="s")

    @functools.partial(
        pl.kernel,
        mesh=mesh,
        out_type=tuple(
            jax.ShapeDtypeStruct((LAT, B // CH, CH), jnp.float32)
            for _ in range(2)),
        scratch_types=[
            pltpu.VMEM((nch, CH), jnp.int32),
            pltpu.VMEM((nch, CH), jnp.int32),
            pltpu.VMEM((LAT, nch, CH), jnp.float32),
            pltpu.VMEM((LAT, nch, CH), jnp.float32),
            pltpu.SemaphoreType.DMA,
        ],
    )
    def k(uidx_hbm, iidx_hbm, *rest):
        slab_refs = rest[:2 * LAT]
        outs = rest[2 * LAT:2 * LAT + 2]
        uidx_v, iidx_v, ru, ri, sem = rest[2 * LAT + 2:]
        wid = lax.axis_index("s") * NC + lax.axis_index("c")
        base = wid * bpw
        for c in range(nch):
            pltpu.sync_copy(uidx_hbm.at[pl.ds(base + c * CH, CH)],
                            uidx_v.at[c])
            pltpu.sync_copy(iidx_hbm.at[pl.ds(base + c * CH, CH)],
                            iidx_v.at[c])
        pending = []
        for t, (buf, idxref) in enumerate(((ru, uidx_v), (ri, iidx_v))):
            issued = []
            for f in range(LAT):
                slab = slab_refs[t * LAT + f]
                for c in range(nch):
                    issued.append(pltpu.async_copy(
                        slab.at[idxref.at[c]], buf.at[f, c], sem))
            for cp in pending:
                cp.wait()
            pending = issued
        for cp in pending:
            cp.wait()
        cols = pl.ds(wid * nch, nch)
        for buf, out in zip((ru, ri), outs):
            pltpu.sync_copy(buf, out.at[:, cols])

    outs = k(user_idx, item_idx, *slabs)
    return [o.reshape(LAT, B) for o in outs]


def _tc_mlp_t(uT, iT, umfT, imfT, W1, b1, W2, b2, W_out, b_out):
    B = uT.shape[1]
    BLK = 2048
    HID = LAT // 2
    W1aT = W1[:LAT].T
    W1bT = W1[LAT:].T
    W2T = W2.T
    b1c = b1.reshape(LAT, 1)
    b2c = b2.reshape(HID, 1)
    wh2 = W_out[:HID, 0].reshape(1, HID)
    wmf = W_out[HID:, 0].reshape(1, LAT)
    bor = b_out.reshape(1, 1)

    def body(u_ref, i_ref, umf_ref, imf_ref, w1a, w1b, b1_, w2, b2_, wh2_,
             wmf_, bo, out_ref):
        mf = umf_ref[...] * imf_ref[...]
        h1 = jnp.maximum(
            jnp.dot(w1a[...], u_ref[...], preferred_element_type=jnp.float32)
            + jnp.dot(w1b[...], i_ref[...], preferred_element_type=jnp.float32)
            + b1_[...], 0.0)
        h2 = jnp.maximum(
            jnp.dot(w2[...], h1, preferred_element_type=jnp.float32)
            + b2_[...], 0.0)
        logit = (jnp.dot(wh2_[...], h2, preferred_element_type=jnp.float32)
                 + jnp.dot(wmf_[...], mf, preferred_element_type=jnp.float32)
                 + bo[...])
        out_ref[...] = jax.nn.sigmoid(logit)

    col = lambda i: (0, i)
    rep = lambda i: (0, 0)
    return pl.pallas_call(
        body,
        grid=(B // BLK,),
        in_specs=[
            pl.BlockSpec((LAT, BLK), col),
            pl.BlockSpec((LAT, BLK), col),
            pl.BlockSpec((LAT, BLK), col),
            pl.BlockSpec((LAT, BLK), col),
            pl.BlockSpec((LAT, LAT), rep),
            pl.BlockSpec((LAT, LAT), rep),
            pl.BlockSpec((LAT, 1), rep),
            pl.BlockSpec((HID, LAT), rep),
            pl.BlockSpec((HID, 1), rep),
            pl.BlockSpec((1, HID), rep),
            pl.BlockSpec((1, LAT), rep),
            pl.BlockSpec((1, 1), rep),
        ],
        out_specs=pl.BlockSpec((1, BLK), col),
        out_shape=jax.ShapeDtypeStruct((1, B), jnp.float32),
    )(uT, iT, umfT, imfT, W1aT, W1bT, b1c, W2T, b2c, wh2, wmf, bor)


def kernel(user_indices, item_indices, emb_user_mlp, emb_item_mlp,
           emb_user_mf, emb_item_mf, W1, b1, W2, b2, W_out, b_out):
    slabs_mlp = _tc_detile(emb_user_mlp, emb_item_mlp)
    uT, iT = _sc_gather_t(user_indices, item_indices, slabs_mlp)
    slabs_mf = _tc_detile(emb_user_mf, emb_item_mf)
    umfT, imfT = _sc_gather_t(user_indices, item_indices, slabs_mf)
    out = _tc_mlp_t(uT, iT, umfT, imfT, W1, b1, W2, b2, W_out, b_out)
    return out.reshape(user_indices.shape[0], 1)


# speedup vs baseline: 1.4578x; 1.0539x over previous
import functools

import jax
import jax.numpy as jnp
from jax import lax
from jax.experimental import pallas as pl
from jax.experimental.pallas import tpu as pltpu
from jax.experimental.pallas import tpu_sc as plsc

LAT = 16
CH = 128


def _tc_detile(*tables):
    V = tables[0].shape[0]
    W = 65536
    grid = (pl.cdiv(V, W),)
    n = len(tables)
    views = [jnp.transpose(t).reshape(LAT, 1, V) for t in tables]

    def body(*refs):
        for s in range(n * LAT):
            refs[n * LAT + s][...] = refs[s][0, 0, :]

    in_specs = []
    for _ in range(n):
        for f in range(LAT):
            in_specs.append(
                pl.BlockSpec((1, 1, W), lambda i, f=f: (f, 0, i)))
    return pl.pallas_call(
        body,
        grid=grid,
        in_specs=in_specs,
        out_specs=[pl.BlockSpec((W,), lambda i: (i,))] * (n * LAT),
        out_shape=[jax.ShapeDtypeStruct((V,), jnp.float32)] * (n * LAT),
    )(*[v for v in views for _ in range(LAT)])


def _sc_gather_t(user_idx, item_idx, slabs):
    B = user_idx.shape[0]
    info = plsc.get_sparse_core_info()
    NC, NS = info.num_cores, info.num_subcores
    NW = NC * NS
    bpw = B // NW
    nch = bpw // CH
    mesh = plsc.VectorSubcoreMesh(core_axis_name="c", subcore_axis_name="s")

    @functools.partial(
        pl.kernel,
        mesh=mesh,
        out_type=tuple(
            jax.ShapeDtypeStruct((LAT, B // CH, CH), jnp.float32)
            for _ in range(2)),
        scratch_types=[
            pltpu.VMEM((nch, CH), jnp.int32),
            pltpu.VMEM((nch, CH), jnp.int32),
            pltpu.VMEM((LAT, nch, CH), jnp.float32),
            pltpu.VMEM((LAT, nch, CH), jnp.float32),
            pltpu.SemaphoreType.DMA,
        ],
    )
    def k(uidx_hbm, iidx_hbm, *rest):
        slab_refs = rest[:2 * LAT]
        outs = rest[2 * LAT:2 * LAT + 2]
        uidx_v, iidx_v, ru, ri, sem = rest[2 * LAT + 2:]
        wid = lax.axis_index("s") * NC + lax.axis_index("c")
        base = wid * bpw
        for c in range(nch):
            pltpu.sync_copy(uidx_hbm.at[pl.ds(base + c * CH, CH)],
                            uidx_v.at[c])
            pltpu.sync_copy(iidx_hbm.at[pl.ds(base + c * CH, CH)],
                            iidx_v.at[c])
        pending = []
        for t, (buf, idxref) in enumerate(((ru, uidx_v), (ri, iidx_v))):
            issued = []
            for f in range(LAT):
                slab = slab_refs[t * LAT + f]
                for c in range(nch):
                    issued.append(pltpu.async_copy(
                        slab.at[idxref.at[c]], buf.at[f, c], sem))
            for cp in pending:
                cp.wait()
            pending = issued
        for cp in pending:
            cp.wait()
        cols = pl.ds(wid * nch, nch)
        for buf, out in zip((ru, ri), outs):
            pltpu.sync_copy(buf, out.at[:, cols])

    outs = k(user_idx, item_idx, *slabs)
    return [o.reshape(LAT, B) for o in outs]


def _tc_mlp_t(uT, iT, umfT, imfT, W1, b1, W2, b2, W_out, b_out):
    B = uT.shape[1]
    BLK = 2048
    HID = LAT // 2
    W1aT = W1[:LAT].T
    W1bT = W1[LAT:].T
    W2T = W2.T
    b1c = b1.reshape(LAT, 1)
    b2c = b2.reshape(HID, 1)
    wh2 = W_out[:HID, 0].reshape(1, HID)
    wmf = W_out[HID:, 0].reshape(1, LAT)
    bor = b_out.reshape(1, 1)

    def body(u_ref, i_ref, umf_ref, imf_ref, w1a, w1b, b1_, w2, b2_, wh2_,
             wmf_, bo, out_ref):
        mf = umf_ref[...] * imf_ref[...]
        h1 = jnp.maximum(
            jnp.dot(w1a[...], u_ref[...], preferred_element_type=jnp.float32)
            + jnp.dot(w1b[...], i_ref[...], preferred_element_type=jnp.float32)
            + b1_[...], 0.0)
        h2 = jnp.maximum(
            jnp.dot(w2[...], h1, preferred_element_type=jnp.float32)
            + b2_[...], 0.0)
        logit = (jnp.dot(wh2_[...], h2, preferred_element_type=jnp.float32)
                 + jnp.dot(wmf_[...], mf, preferred_element_type=jnp.float32)
                 + bo[...])
        out_ref[...] = jax.nn.sigmoid(logit)

    col = lambda i: (0, i)
    rep = lambda i: (0, 0)
    return pl.pallas_call(
        body,
        grid=(B // BLK,),
        in_specs=[
            pl.BlockSpec((LAT, BLK), col),
            pl.BlockSpec((LAT, BLK), col),
            pl.BlockSpec((LAT, BLK), col),
            pl.BlockSpec((LAT, BLK), col),
            pl.BlockSpec((LAT, LAT), rep),
            pl.BlockSpec((LAT, LAT), rep),
            pl.BlockSpec((LAT, 1), rep),
            pl.BlockSpec((HID, LAT), rep),
            pl.BlockSpec((HID, 1), rep),
            pl.BlockSpec((1, HID), rep),
            pl.BlockSpec((1, LAT), rep),
            pl.BlockSpec((1, 1), rep),
        ],
        out_specs=pl.BlockSpec((1, BLK), col),
        out_shape=jax.ShapeDtypeStruct((1, B), jnp.float32),
    )(uT, iT, umfT, imfT, W1aT, W1bT, b1c, W2T, b2c, wh2, wmf, bor)


def kernel(user_indices, item_indices, emb_user_mlp, emb_item_mlp,
           emb_user_mf, emb_item_mf, W1, b1, W2, b2, W_out, b_out):
    slabs_mlp = _tc_detile(emb_user_mlp, emb_item_mlp)
    uT, iT = _sc_gather_t(user_indices, item_indices, slabs_mlp)
    slabs_mf = _tc_detile(emb_user_mf, emb_item_mf)
    umfT, imfT = _sc_gather_t(user_indices, item_indices, slabs_mf)
    out = _tc_mlp_t(uT, iT, umfT, imfT, W1, b1, W2, b2, W_out, b_out)
    return out.reshape(user_indices.shape[0], 1)


# speedup vs baseline: 1.4585x; 1.0005x over previous
import functools

import jax
import jax.numpy as jnp
from jax import lax
from jax.experimental import pallas as pl
from jax.experimental.pallas import tpu as pltpu
from jax.experimental.pallas import tpu_sc as plsc

LAT = 16
CH = 128


def _tc_detile(*tables):
    V = tables[0].shape[0]
    W = 65536
    grid = (pl.cdiv(V, W),)
    n = len(tables)
    views = [jnp.transpose(t).reshape(LAT, 1, V) for t in tables]

    def body(*refs):
        for s in range(n * LAT):
            refs[n * LAT + s][...] = refs[s][0, 0, :]

    in_specs = []
    for _ in range(n):
        for f in range(LAT):
            in_specs.append(
                pl.BlockSpec((1, 1, W), lambda i, f=f: (f, 0, i)))
    return pl.pallas_call(
        body,
        grid=grid,
        in_specs=in_specs,
        out_specs=[pl.BlockSpec((W,), lambda i: (i,))] * (n * LAT),
        out_shape=[jax.ShapeDtypeStruct((V,), jnp.float32)] * (n * LAT),
    )(*[v for v in views for _ in range(LAT)])


def _sc_gather_t(user_idx, item_idx, slabs):
    B = user_idx.shape[0]
    info = plsc.get_sparse_core_info()
    NC, NS = info.num_cores, info.num_subcores
    NW = NC * NS
    bpw = B // NW
    nch = bpw // CH
    mesh = plsc.VectorSubcoreMesh(core_axis_name="c", subcore_axis_name="s")

    @functools.partial(
        pl.kernel,
        mesh=mesh,
        out_type=tuple(
            jax.ShapeDtypeStruct((LAT, B // CH, CH), jnp.float32)
            for _ in range(2)),
        scratch_types=[
            pltpu.VMEM((nch, CH), jnp.int32),
            pltpu.VMEM((nch, CH), jnp.int32),
            pltpu.VMEM((LAT, nch, CH), jnp.float32),
            pltpu.VMEM((LAT, nch, CH), jnp.float32),
            pltpu.SemaphoreType.DMA,
        ],
    )
    def k(uidx_hbm, iidx_hbm, *rest):
        slab_refs = rest[:2 * LAT]
        outs = rest[2 * LAT:2 * LAT + 2]
        uidx_v, iidx_v, ru, ri, sem = rest[2 * LAT + 2:]
        wid = lax.axis_index("s") * NC + lax.axis_index("c")
        base = wid * bpw
        for c in range(nch):
            pltpu.sync_copy(uidx_hbm.at[pl.ds(base + c * CH, CH)],
                            uidx_v.at[c])
            pltpu.sync_copy(iidx_hbm.at[pl.ds(base + c * CH, CH)],
                            iidx_v.at[c])
        pending = []
        for t, (buf, idxref) in enumerate(((ru, uidx_v), (ri, iidx_v))):
            issued = []
            for f in range(LAT):
                slab = slab_refs[t * LAT + f]
                for c in range(nch):
                    issued.append(pltpu.async_copy(
                        slab.at[idxref.at[c]], buf.at[f, c], sem))
            for cp in pending:
                cp.wait()
            pending = issued
        for cp in pending:
            cp.wait()
        cols = pl.ds(wid * nch, nch)
        for buf, out in zip((ru, ri), outs):
            pltpu.sync_copy(buf, out.at[:, cols])

    outs = k(user_idx, item_idx, *slabs)
    return [o.reshape(LAT, B) for o in outs]


def _tc_mlp_t(uT, iT, umfT, imfT, W1, b1, W2, b2, W_out, b_out):
    B = uT.shape[1]
    BLK = 2048
    HID = LAT // 2
    W1aT = W1[:LAT].T
    W1bT = W1[LAT:].T
    W2T = W2.T
    b1c = b1.reshape(LAT, 1)
    b2c = b2.reshape(HID, 1)
    wh2 = W_out[:HID, 0].reshape(1, HID)
    wmf = W_out[HID:, 0].reshape(1, LAT)
    bor = b_out.reshape(1, 1)

    def body(u_ref, i_ref, umf_ref, imf_ref, w1a, w1b, b1_, w2, b2_, wh2_,
             wmf_, bo, out_ref):
        mf = umf_ref[...] * imf_ref[...]
        h1 = jnp.maximum(
            jnp.dot(w1a[...], u_ref[...], preferred_element_type=jnp.float32)
            + jnp.dot(w1b[...], i_ref[...], preferred_element_type=jnp.float32)
            + b1_[...], 0.0)
        h2 = jnp.maximum(
            jnp.dot(w2[...], h1, preferred_element_type=jnp.float32)
            + b2_[...], 0.0)
        logit = (jnp.dot(wh2_[...], h2, preferred_element_type=jnp.float32)
                 + jnp.dot(wmf_[...], mf, preferred_element_type=jnp.float32)
                 + bo[...])
        out_ref[...] = jax.nn.sigmoid(logit)

    col = lambda i: (0, i)
    rep = lambda i: (0, 0)
    return pl.pallas_call(
        body,
        grid=(B // BLK,),
        in_specs=[
            pl.BlockSpec((LAT, BLK), col),
            pl.BlockSpec((LAT, BLK), col),
            pl.BlockSpec((LAT, BLK), col),
            pl.BlockSpec((LAT, BLK), col),
            pl.BlockSpec((LAT, LAT), rep),
            pl.BlockSpec((LAT, LAT), rep),
            pl.BlockSpec((LAT, 1), rep),
            pl.BlockSpec((HID, LAT), rep),
            pl.BlockSpec((HID, 1), rep),
            pl.BlockSpec((1, HID), rep),
            pl.BlockSpec((1, LAT), rep),
            pl.BlockSpec((1, 1), rep),
        ],
        out_specs=pl.BlockSpec((1, BLK), col),
        out_shape=jax.ShapeDtypeStruct((1, B), jnp.float32),
    )(uT, iT, umfT, imfT, W1aT, W1bT, b1c, W2T, b2c, wh2, wmf, bor)


def kernel(user_indices, item_indices, emb_user_mlp, emb_item_mlp,
           emb_user_mf, emb_item_mf, W1, b1, W2, b2, W_out, b_out):
    slabs_mlp = _tc_detile(emb_user_mlp, emb_item_mlp)
    slabs_mf = _tc_detile(emb_user_mf, emb_item_mf)
    uT, iT = _sc_gather_t(user_indices, item_indices, slabs_mlp)
    umfT, imfT = _sc_gather_t(user_indices, item_indices, slabs_mf)
    out = _tc_mlp_t(uT, iT, umfT, imfT, W1, b1, W2, b2, W_out, b_out)
    return out.reshape(user_indices.shape[0], 1)
